# Initial kernel scaffold; baseline (speedup 1.0000x reference)
import jax
import jax.numpy as jnp
from jax.experimental import pallas as pl


def kernel(x, edge_index, W1, b1, W2, b2, W3, b3):
    raise NotImplementedError("write your pallas kernel here")


# speedup vs baseline: 8.2596x; 8.2596x over previous
import functools

import jax
import jax.numpy as jnp
from jax import lax
from jax.experimental import pallas as pl
from jax.experimental.pallas import tpu as pltpu
from jax.experimental.pallas import tpu_sc as plsc

N_NODES = 10000
DIM_IN = 128
DIM_H = 256
DIM_OUT = 64

NC = 2
NS = 16
NW = NC * NS
L = 16

N_PAD = 10240
ROWS_PER_TILE = N_PAD // NS
HALF = DIM_H // 2
CB = 128


def _hist_body(eh, dst_hbm, out_hbm, dst_v, hist_v):
    c = lax.axis_index("c")
    s = lax.axis_index("s")
    wid = s * NC + c
    pltpu.sync_copy(dst_hbm.at[wid], dst_v)
    zeros16 = jnp.zeros((L,), jnp.float32)

    def zbody(g, carry):
        hist_v[pl.ds(g * L, L)] = zeros16
        return carry

    lax.fori_loop(0, N_PAD // L, zbody, 0)
    ones16 = jnp.ones((L,), jnp.float32)

    def body(g, carry):
        idx = dst_v[pl.ds(g * L, L)]
        plsc.addupdate_scatter(hist_v, [idx], ones16)
        return carry

    lax.fori_loop(0, eh // L, body, 0)
    pltpu.sync_copy(hist_v, out_hbm.at[wid])


def _make_hist(eh):
    return pl.kernel(
        functools.partial(_hist_body, eh),
        out_type=jax.ShapeDtypeStruct((NW, N_PAD), jnp.float32),
        mesh=plsc.VectorSubcoreMesh(core_axis_name="c", subcore_axis_name="s"),
        compiler_params=pltpu.CompilerParams(needs_layout_passes=False),
        scratch_types=[
            pltpu.VMEM((eh,), jnp.int32),
            pltpu.VMEM((N_PAD,), jnp.float32),
        ],
    )


KSUP = 16
ESUP = KSUP * CB


def _mp_body(nsup, hs_hbm, src_hbm, dst_hbm, out_hbm, src_buf, dst_buf, rows_v, agg_sh, sem):
    c = lax.axis_index("c")
    s = lax.axis_index("s")
    r0 = s * ROWS_PER_TILE
    pltpu.sync_copy(hs_hbm.at[c, pl.ds(r0, ROWS_PER_TILE)],
                    agg_sh.at[pl.ds(r0, ROWS_PER_TILE)])
    plsc.subcore_barrier()

    def outer(o, carry):
        pltpu.sync_copy(src_hbm.at[s, o], src_buf)
        pltpu.sync_copy(dst_hbm.at[s, o], dst_buf)

        def inner(k, carry2):
            pltpu.async_copy(hs_hbm.at[c].at[src_buf.at[k]], rows_v, sem).wait()
            pltpu.sync_copy(rows_v, agg_sh.at[dst_buf.at[k]], add=True)
            return carry2

        lax.fori_loop(0, KSUP, inner, 0)
        return carry

    lax.fori_loop(0, nsup, outer, 0)
    plsc.subcore_barrier()
    pltpu.sync_copy(agg_sh.at[pl.ds(r0, ROWS_PER_TILE)],
                    out_hbm.at[c, pl.ds(r0, ROWS_PER_TILE)])


def _make_mp(nsup):
    return pl.kernel(
        functools.partial(_mp_body, nsup),
        out_type=jax.ShapeDtypeStruct((NC, N_PAD, HALF), jnp.float32),
        mesh=plsc.VectorSubcoreMesh(core_axis_name="c", subcore_axis_name="s"),
        compiler_params=pltpu.CompilerParams(needs_layout_passes=False),
        scratch_types=[
            pltpu.VMEM((KSUP, CB), jnp.int32),
            pltpu.VMEM((KSUP, CB), jnp.int32),
            pltpu.VMEM((CB, HALF), jnp.float32),
            pltpu.VMEM_SHARED((N_PAD, HALF), jnp.float32),
            pltpu.SemaphoreType.DMA,
        ],
    )


def _dinv_from(deg_ref):
    dsum = jnp.sum(deg_ref[...], axis=0) + 1.0
    return lax.rsqrt(dsum)[:, None]


def _lin1_tc(x_ref, w_ref, deg_ref, out_ref):
    dinv = _dinv_from(deg_ref)
    h = jnp.dot(x_ref[...], w_ref[...], preferred_element_type=jnp.float32)
    hs = h * dinv
    out_ref[0] = hs[:, :HALF]
    out_ref[1] = hs[:, HALF:]


def _mid_tc(agg_ref, w_ref, b_ref, deg_ref, out_ref):
    dinv = _dinv_from(deg_ref)
    hl = jnp.maximum(agg_ref[0] * dinv + b_ref[:, :HALF], 0.0)
    hr = jnp.maximum(agg_ref[1] * dinv + b_ref[:, HALF:], 0.0)
    h2 = (jnp.dot(hl, w_ref[:HALF, :], preferred_element_type=jnp.float32)
          + jnp.dot(hr, w_ref[HALF:, :], preferred_element_type=jnp.float32))
    hs = h2 * dinv
    out_ref[0] = hs[:, :HALF]
    out_ref[1] = hs[:, HALF:]


def _out_tc(agg_ref, b2_ref, w3_ref, b3_ref, deg_ref, out_ref):
    dinv = _dinv_from(deg_ref)
    hl = jnp.maximum(agg_ref[0] * dinv + b2_ref[:, :HALF], 0.0)
    hr = jnp.maximum(agg_ref[1] * dinv + b2_ref[:, HALF:], 0.0)
    logits = (jnp.dot(hl, w3_ref[:HALF, :], preferred_element_type=jnp.float32)
              + jnp.dot(hr, w3_ref[HALF:, :], preferred_element_type=jnp.float32)
              + b3_ref[...])
    m = jnp.max(logits, axis=1, keepdims=True)
    sh = logits - m
    lse = jnp.log(jnp.sum(jnp.exp(sh), axis=1, keepdims=True))
    out_ref[...] = sh - lse


BN = 1024


def _lin1_call(xp, w1, deg_parts):
    return pl.pallas_call(
        _lin1_tc,
        grid=(N_PAD // BN,),
        in_specs=[
            pl.BlockSpec((BN, DIM_IN), lambda i: (i, 0)),
            pl.BlockSpec((DIM_IN, DIM_H), lambda i: (0, 0)),
            pl.BlockSpec((NW, BN), lambda i: (0, i)),
        ],
        out_specs=pl.BlockSpec((NC, BN, HALF), lambda i: (0, i, 0)),
        out_shape=jax.ShapeDtypeStruct((NC, N_PAD, HALF), jnp.float32),
    )(xp, w1, deg_parts)


def _mid_call(agg, w2, b1r, deg_parts):
    return pl.pallas_call(
        _mid_tc,
        grid=(N_PAD // BN,),
        in_specs=[
            pl.BlockSpec((NC, BN, HALF), lambda i: (0, i, 0)),
            pl.BlockSpec((DIM_H, DIM_H), lambda i: (0, 0)),
            pl.BlockSpec((1, DIM_H), lambda i: (0, 0)),
            pl.BlockSpec((NW, BN), lambda i: (0, i)),
        ],
        out_specs=pl.BlockSpec((NC, BN, HALF), lambda i: (0, i, 0)),
        out_shape=jax.ShapeDtypeStruct((NC, N_PAD, HALF), jnp.float32),
    )(agg, w2, b1r, deg_parts)


def _out_call(agg, b2r, w3, b3r, deg_parts):
    return pl.pallas_call(
        _out_tc,
        grid=(N_PAD // BN,),
        in_specs=[
            pl.BlockSpec((NC, BN, HALF), lambda i: (0, i, 0)),
            pl.BlockSpec((1, DIM_H), lambda i: (0, 0)),
            pl.BlockSpec((DIM_H, DIM_OUT), lambda i: (0, 0)),
            pl.BlockSpec((1, DIM_OUT), lambda i: (0, 0)),
            pl.BlockSpec((NW, BN), lambda i: (0, i)),
        ],
        out_specs=pl.BlockSpec((BN, DIM_OUT), lambda i: (i, 0)),
        out_shape=jax.ShapeDtypeStruct((N_PAD, DIM_OUT), jnp.float32),
    )(agg, b2r, w3, b3r, deg_parts)


def kernel(x, edge_index, W1, b1, W2, b2, W3, b3):
    e = edge_index.shape[1]
    src = edge_index[0].astype(jnp.int32)
    dst = edge_index[1].astype(jnp.int32)

    eh = e // NW
    deg_parts = _make_hist(eh)(dst.reshape(NW, eh))

    em = -(-e // (NS * ESUP)) * ESUP
    pad = NS * em - e
    fill = jnp.full((pad,), N_NODES, jnp.int32)
    srcp = jnp.concatenate([src, fill]).reshape(NS, em // ESUP, KSUP, CB)
    dstp = jnp.concatenate([dst, fill]).reshape(NS, em // ESUP, KSUP, CB)
    mp = _make_mp(em // ESUP)

    xp = jnp.pad(x, ((0, N_PAD - N_NODES), (0, 0)))
    b1r = b1.reshape(1, DIM_H)
    b2r = b2.reshape(1, DIM_H)
    b3r = b3.reshape(1, DIM_OUT)

    hs1 = _lin1_call(xp, W1, deg_parts)
    agg1 = mp(hs1, srcp, dstp)
    hs2 = _mid_call(agg1, W2, b1r, deg_parts)
    agg2 = mp(hs2, srcp, dstp)
    return _out_call(agg2, b2r, W3, b3r, deg_parts)[:N_NODES]


# speedup vs baseline: 9.4448x; 1.1435x over previous
import functools

import jax
import jax.numpy as jnp
from jax import lax
from jax.experimental import pallas as pl
from jax.experimental.pallas import tpu as pltpu
from jax.experimental.pallas import tpu_sc as plsc

N_NODES = 10000
DIM_IN = 128
DIM_H = 256
DIM_OUT = 64

NC = 2
NS = 16
NW = NC * NS
L = 16

N_PAD = 10240
ROWS_PER_TILE = N_PAD // NS
HALF = DIM_H // 2
CB = 128


def _hist_body(eh, dst_hbm, out_hbm, dst_v, hist_v):
    c = lax.axis_index("c")
    s = lax.axis_index("s")
    wid = s * NC + c
    pltpu.sync_copy(dst_hbm.at[wid], dst_v)
    zeros16 = jnp.zeros((L,), jnp.float32)

    def zbody(g, carry):
        hist_v[pl.ds(g * L, L)] = zeros16
        return carry

    lax.fori_loop(0, N_PAD // L, zbody, 0)
    ones16 = jnp.ones((L,), jnp.float32)

    def body(g, carry):
        idx = dst_v[pl.ds(g * L, L)]
        plsc.addupdate_scatter(hist_v, [idx], ones16)
        return carry

    lax.fori_loop(0, eh // L, body, 0)
    pltpu.sync_copy(hist_v, out_hbm.at[wid])


def _make_hist(eh):
    return pl.kernel(
        functools.partial(_hist_body, eh),
        out_type=jax.ShapeDtypeStruct((NW, N_PAD), jnp.float32),
        mesh=plsc.VectorSubcoreMesh(core_axis_name="c", subcore_axis_name="s"),
        compiler_params=pltpu.CompilerParams(needs_layout_passes=False),
        scratch_types=[
            pltpu.VMEM((eh,), jnp.int32),
            pltpu.VMEM((N_PAD,), jnp.float32),
        ],
    )


KSUP = 16
ESUP = KSUP * CB


def _mp_body(nsup, hs_hbm, src_hbm, dst_hbm, out_hbm,
             src_buf, dst_buf, rows_v, agg_sh, g0, g1, s0, s1):
    c = lax.axis_index("c")
    s = lax.axis_index("s")
    r0 = s * ROWS_PER_TILE
    nblk = nsup * KSUP
    pltpu.sync_copy(hs_hbm.at[c, pl.ds(r0, ROWS_PER_TILE)],
                    agg_sh.at[pl.ds(r0, ROWS_PER_TILE)])
    plsc.subcore_barrier()
    hsc = hs_hbm.at[c]

    def idx_ref(buf, k):
        return buf.at[(k // KSUP) % 2, k % KSUP]

    pltpu.sync_copy(src_hbm.at[s, 0], src_buf.at[0])
    pltpu.sync_copy(dst_hbm.at[s, 0], dst_buf.at[0])
    pltpu.async_copy(hsc.at[idx_ref(src_buf, 0)], rows_v.at[0], g0)

    def chunk(k, b, gsem_b, gsem_o, ssem_b, ssem_o):
        pltpu.make_async_copy(hsc.at[idx_ref(src_buf, k)], rows_v.at[b],
                              gsem_b).wait()
        pltpu.async_copy(rows_v.at[b], agg_sh.at[idx_ref(dst_buf, k)],
                         ssem_b, add=True)
        kn = k + 1

        @pl.when(kn < nblk)
        def _():
            @pl.when((kn % KSUP == 0) & (kn // KSUP > 0))
            def _():
                pltpu.sync_copy(src_hbm.at[s, kn // KSUP],
                                src_buf.at[(kn // KSUP) % 2])
                pltpu.sync_copy(dst_hbm.at[s, kn // KSUP],
                                dst_buf.at[(kn // KSUP) % 2])

            @pl.when(k > 0)
            def _():
                pltpu.make_async_copy(rows_v.at[1 - b],
                                      agg_sh.at[idx_ref(dst_buf, k - 1)],
                                      ssem_o).wait()

            pltpu.async_copy(hsc.at[idx_ref(src_buf, kn)], rows_v.at[1 - b],
                             gsem_o)

    def pair(p, carry):
        chunk(2 * p, 0, g0, g1, s0, s1)
        chunk(2 * p + 1, 1, g1, g0, s1, s0)
        return carry

    lax.fori_loop(0, nblk // 2, pair, 0)
    pltpu.make_async_copy(rows_v.at[0], agg_sh.at[idx_ref(dst_buf, nblk - 2)],
                          s0).wait()
    pltpu.make_async_copy(rows_v.at[1], agg_sh.at[idx_ref(dst_buf, nblk - 1)],
                          s1).wait()
    plsc.subcore_barrier()
    pltpu.sync_copy(agg_sh.at[pl.ds(r0, ROWS_PER_TILE)],
                    out_hbm.at[c, pl.ds(r0, ROWS_PER_TILE)])


def _make_mp(nsup):
    return pl.kernel(
        functools.partial(_mp_body, nsup),
        out_type=jax.ShapeDtypeStruct((NC, N_PAD, HALF), jnp.float32),
        mesh=plsc.VectorSubcoreMesh(core_axis_name="c", subcore_axis_name="s"),
        compiler_params=pltpu.CompilerParams(needs_layout_passes=False),
        scratch_types=[
            pltpu.VMEM((2, KSUP, CB), jnp.int32),
            pltpu.VMEM((2, KSUP, CB), jnp.int32),
            pltpu.VMEM((2, CB, HALF), jnp.float32),
            pltpu.VMEM_SHARED((N_PAD, HALF), jnp.float32),
            pltpu.SemaphoreType.DMA,
            pltpu.SemaphoreType.DMA,
            pltpu.SemaphoreType.DMA,
            pltpu.SemaphoreType.DMA,
        ],
    )


def _dinv_from(deg_ref):
    dsum = jnp.sum(deg_ref[...], axis=0) + 1.0
    return lax.rsqrt(dsum)[:, None]


def _lin1_tc(x_ref, w_ref, deg_ref, out_ref):
    dinv = _dinv_from(deg_ref)
    h = jnp.dot(x_ref[...], w_ref[...], preferred_element_type=jnp.float32)
    hs = h * dinv
    out_ref[0] = hs[:, :HALF]
    out_ref[1] = hs[:, HALF:]


def _mid_tc(agg_ref, w_ref, b_ref, deg_ref, out_ref):
    dinv = _dinv_from(deg_ref)
    hl = jnp.maximum(agg_ref[0] * dinv + b_ref[:, :HALF], 0.0)
    hr = jnp.maximum(agg_ref[1] * dinv + b_ref[:, HALF:], 0.0)
    h2 = (jnp.dot(hl, w_ref[:HALF, :], preferred_element_type=jnp.float32)
          + jnp.dot(hr, w_ref[HALF:, :], preferred_element_type=jnp.float32))
    hs = h2 * dinv
    out_ref[0] = hs[:, :HALF]
    out_ref[1] = hs[:, HALF:]


def _out_tc(agg_ref, b2_ref, w3_ref, b3_ref, deg_ref, out_ref):
    dinv = _dinv_from(deg_ref)
    hl = jnp.maximum(agg_ref[0] * dinv + b2_ref[:, :HALF], 0.0)
    hr = jnp.maximum(agg_ref[1] * dinv + b2_ref[:, HALF:], 0.0)
    logits = (jnp.dot(hl, w3_ref[:HALF, :], preferred_element_type=jnp.float32)
              + jnp.dot(hr, w3_ref[HALF:, :], preferred_element_type=jnp.float32)
              + b3_ref[...])
    m = jnp.max(logits, axis=1, keepdims=True)
    sh = logits - m
    lse = jnp.log(jnp.sum(jnp.exp(sh), axis=1, keepdims=True))
    out_ref[...] = sh - lse


BN = 1024


def _lin1_call(xp, w1, deg_parts):
    return pl.pallas_call(
        _lin1_tc,
        grid=(N_PAD // BN,),
        in_specs=[
            pl.BlockSpec((BN, DIM_IN), lambda i: (i, 0)),
            pl.BlockSpec((DIM_IN, DIM_H), lambda i: (0, 0)),
            pl.BlockSpec((NW, BN), lambda i: (0, i)),
        ],
        out_specs=pl.BlockSpec((NC, BN, HALF), lambda i: (0, i, 0)),
        out_shape=jax.ShapeDtypeStruct((NC, N_PAD, HALF), jnp.float32),
    )(xp, w1, deg_parts)


def _mid_call(agg, w2, b1r, deg_parts):
    return pl.pallas_call(
        _mid_tc,
        grid=(N_PAD // BN,),
        in_specs=[
            pl.BlockSpec((NC, BN, HALF), lambda i: (0, i, 0)),
            pl.BlockSpec((DIM_H, DIM_H), lambda i: (0, 0)),
            pl.BlockSpec((1, DIM_H), lambda i: (0, 0)),
            pl.BlockSpec((NW, BN), lambda i: (0, i)),
        ],
        out_specs=pl.BlockSpec((NC, BN, HALF), lambda i: (0, i, 0)),
        out_shape=jax.ShapeDtypeStruct((NC, N_PAD, HALF), jnp.float32),
    )(agg, w2, b1r, deg_parts)


def _out_call(agg, b2r, w3, b3r, deg_parts):
    return pl.pallas_call(
        _out_tc,
        grid=(N_PAD // BN,),
        in_specs=[
            pl.BlockSpec((NC, BN, HALF), lambda i: (0, i, 0)),
            pl.BlockSpec((1, DIM_H), lambda i: (0, 0)),
            pl.BlockSpec((DIM_H, DIM_OUT), lambda i: (0, 0)),
            pl.BlockSpec((1, DIM_OUT), lambda i: (0, 0)),
            pl.BlockSpec((NW, BN), lambda i: (0, i)),
        ],
        out_specs=pl.BlockSpec((BN, DIM_OUT), lambda i: (i, 0)),
        out_shape=jax.ShapeDtypeStruct((N_PAD, DIM_OUT), jnp.float32),
    )(agg, b2r, w3, b3r, deg_parts)


def kernel(x, edge_index, W1, b1, W2, b2, W3, b3):
    e = edge_index.shape[1]
    src = edge_index[0].astype(jnp.int32)
    dst = edge_index[1].astype(jnp.int32)

    eh = e // NW
    deg_parts = _make_hist(eh)(dst.reshape(NW, eh))

    em = -(-e // (NS * ESUP)) * ESUP
    pad = NS * em - e
    fill = jnp.full((pad,), N_NODES, jnp.int32)
    srcp = jnp.concatenate([src, fill]).reshape(NS, em // ESUP, KSUP, CB)
    dstp = jnp.concatenate([dst, fill]).reshape(NS, em // ESUP, KSUP, CB)
    mp = _make_mp(em // ESUP)

    xp = jnp.pad(x, ((0, N_PAD - N_NODES), (0, 0)))
    b1r = b1.reshape(1, DIM_H)
    b2r = b2.reshape(1, DIM_H)
    b3r = b3.reshape(1, DIM_OUT)

    hs1 = _lin1_call(xp, W1, deg_parts)
    agg1 = mp(hs1, srcp, dstp)
    hs2 = _mid_call(agg1, W2, b1r, deg_parts)
    agg2 = mp(hs2, srcp, dstp)
    return _out_call(agg2, b2r, W3, b3r, deg_parts)[:N_NODES]


# speedup vs baseline: 10.2114x; 1.0812x over previous
import functools

import jax
import jax.numpy as jnp
from jax import lax
from jax.experimental import pallas as pl
from jax.experimental.pallas import tpu as pltpu
from jax.experimental.pallas import tpu_sc as plsc

N_NODES = 10000
DIM_IN = 128
DIM_H = 256
DIM_OUT = 64

NC = 2
NS = 16
NW = NC * NS
L = 16

N_PAD = 10240
ROWS_PER_TILE = N_PAD // NS
HALF = DIM_H // 2
CB = 64


def _hist_body(eh, dst_hbm, out_hbm, dst_v, hist_v):
    c = lax.axis_index("c")
    s = lax.axis_index("s")
    wid = s * NC + c
    pltpu.sync_copy(dst_hbm.at[wid], dst_v)
    zeros16 = jnp.zeros((L,), jnp.float32)

    def zbody(g, carry):
        hist_v[pl.ds(g * L, L)] = zeros16
        return carry

    lax.fori_loop(0, N_PAD // L, zbody, 0)
    ones16 = jnp.ones((L,), jnp.float32)

    def body(g, carry):
        idx = dst_v[pl.ds(g * L, L)]
        plsc.addupdate_scatter(hist_v, [idx], ones16)
        return carry

    lax.fori_loop(0, eh // L, body, 0)
    pltpu.sync_copy(hist_v, out_hbm.at[wid])


def _make_hist(eh):
    return pl.kernel(
        functools.partial(_hist_body, eh),
        out_type=jax.ShapeDtypeStruct((NW, N_PAD), jnp.float32),
        mesh=plsc.VectorSubcoreMesh(core_axis_name="c", subcore_axis_name="s"),
        compiler_params=pltpu.CompilerParams(needs_layout_passes=False),
        scratch_types=[
            pltpu.VMEM((eh,), jnp.int32),
            pltpu.VMEM((N_PAD,), jnp.float32),
        ],
    )


KSUP = 32
ESUP = KSUP * CB
NBUF = 4
GAHEAD = 2


def _mp_body(nsup, hs_hbm, src_hbm, dst_hbm, out_hbm,
             src_buf, dst_buf, rows_v, agg_sh, *sems):
    gsem = sems[:NBUF]
    ssem = sems[NBUF:]
    c = lax.axis_index("c")
    s = lax.axis_index("s")
    r0 = s * ROWS_PER_TILE
    nblk = nsup * KSUP
    pltpu.sync_copy(hs_hbm.at[c, pl.ds(r0, ROWS_PER_TILE)],
                    agg_sh.at[pl.ds(r0, ROWS_PER_TILE)])
    plsc.subcore_barrier()
    hsc = hs_hbm.at[c]

    def idx_ref(buf, k):
        return buf.at[(k // KSUP) % 2, k % KSUP]

    def issue_gather(k, b):
        pltpu.async_copy(hsc.at[idx_ref(src_buf, k)], rows_v.at[b], gsem[b])

    def load_super(sup):
        pltpu.sync_copy(src_hbm.at[s, sup], src_buf.at[sup % 2])
        pltpu.sync_copy(dst_hbm.at[s, sup], dst_buf.at[sup % 2])

    load_super(0)
    for k0 in range(GAHEAD):
        issue_gather(k0, k0)

    def chunk(k, b):
        pltpu.make_async_copy(hsc.at[idx_ref(src_buf, k)], rows_v.at[b],
                              gsem[b]).wait()
        pltpu.async_copy(rows_v.at[b], agg_sh.at[idx_ref(dst_buf, k)],
                         ssem[b], add=True)
        kg = k + GAHEAD
        bg = (b + GAHEAD) % NBUF

        @pl.when(kg < nblk)
        def _():
            @pl.when((kg % KSUP == 0) & (kg // KSUP > 0))
            def _():
                load_super_dyn(kg // KSUP)

            @pl.when(kg >= NBUF)
            def _():
                pltpu.make_async_copy(rows_v.at[bg],
                                      agg_sh.at[idx_ref(dst_buf, kg - NBUF)],
                                      ssem[bg]).wait()

            issue_gather_dyn(k, kg, bg)

    def load_super_dyn(sup):
        pltpu.sync_copy(src_hbm.at[s, sup], src_buf.at[sup % 2])
        pltpu.sync_copy(dst_hbm.at[s, sup], dst_buf.at[sup % 2])

    def issue_gather_dyn(k, kg, bg):
        pltpu.async_copy(hsc.at[idx_ref(src_buf, kg)], rows_v.at[bg], gsem[bg])

    def group(p, carry):
        for b in range(NBUF):
            chunk(NBUF * p + b, b)
        return carry

    lax.fori_loop(0, nblk // NBUF, group, 0)
    for d in range(NBUF):
        k = nblk - NBUF + d
        pltpu.make_async_copy(rows_v.at[k % NBUF],
                              agg_sh.at[idx_ref(dst_buf, k)],
                              ssem[k % NBUF]).wait()
    plsc.subcore_barrier()
    pltpu.sync_copy(agg_sh.at[pl.ds(r0, ROWS_PER_TILE)],
                    out_hbm.at[c, pl.ds(r0, ROWS_PER_TILE)])


def _make_mp(nsup):
    return pl.kernel(
        functools.partial(_mp_body, nsup),
        out_type=jax.ShapeDtypeStruct((NC, N_PAD, HALF), jnp.float32),
        mesh=plsc.VectorSubcoreMesh(core_axis_name="c", subcore_axis_name="s"),
        compiler_params=pltpu.CompilerParams(needs_layout_passes=False),
        scratch_types=[
            pltpu.VMEM((2, KSUP, CB), jnp.int32),
            pltpu.VMEM((2, KSUP, CB), jnp.int32),
            pltpu.VMEM((NBUF, CB, HALF), jnp.float32),
            pltpu.VMEM_SHARED((N_PAD, HALF), jnp.float32),
        ] + [pltpu.SemaphoreType.DMA] * (2 * NBUF),
    )


def _dinv_from(deg_ref):
    dsum = jnp.sum(deg_ref[...], axis=0) + 1.0
    return lax.rsqrt(dsum)[:, None]


def _lin1_tc(x_ref, w_ref, deg_ref, out_ref):
    dinv = _dinv_from(deg_ref)
    h = jnp.dot(x_ref[...], w_ref[...], preferred_element_type=jnp.float32)
    hs = h * dinv
    out_ref[0] = hs[:, :HALF]
    out_ref[1] = hs[:, HALF:]


def _mid_tc(agg_ref, w_ref, b_ref, deg_ref, out_ref):
    dinv = _dinv_from(deg_ref)
    hl = jnp.maximum(agg_ref[0] * dinv + b_ref[:, :HALF], 0.0)
    hr = jnp.maximum(agg_ref[1] * dinv + b_ref[:, HALF:], 0.0)
    h2 = (jnp.dot(hl, w_ref[:HALF, :], preferred_element_type=jnp.float32)
          + jnp.dot(hr, w_ref[HALF:, :], preferred_element_type=jnp.float32))
    hs = h2 * dinv
    out_ref[0] = hs[:, :HALF]
    out_ref[1] = hs[:, HALF:]


def _out_tc(agg_ref, b2_ref, w3_ref, b3_ref, deg_ref, out_ref):
    dinv = _dinv_from(deg_ref)
    hl = jnp.maximum(agg_ref[0] * dinv + b2_ref[:, :HALF], 0.0)
    hr = jnp.maximum(agg_ref[1] * dinv + b2_ref[:, HALF:], 0.0)
    logits = (jnp.dot(hl, w3_ref[:HALF, :], preferred_element_type=jnp.float32)
              + jnp.dot(hr, w3_ref[HALF:, :], preferred_element_type=jnp.float32)
              + b3_ref[...])
    m = jnp.max(logits, axis=1, keepdims=True)
    sh = logits - m
    lse = jnp.log(jnp.sum(jnp.exp(sh), axis=1, keepdims=True))
    out_ref[...] = sh - lse


BN = 1024


def _lin1_call(xp, w1, deg_parts):
    return pl.pallas_call(
        _lin1_tc,
        grid=(N_PAD // BN,),
        in_specs=[
            pl.BlockSpec((BN, DIM_IN), lambda i: (i, 0)),
            pl.BlockSpec((DIM_IN, DIM_H), lambda i: (0, 0)),
            pl.BlockSpec((NW, BN), lambda i: (0, i)),
        ],
        out_specs=pl.BlockSpec((NC, BN, HALF), lambda i: (0, i, 0)),
        out_shape=jax.ShapeDtypeStruct((NC, N_PAD, HALF), jnp.float32),
    )(xp, w1, deg_parts)


def _mid_call(agg, w2, b1r, deg_parts):
    return pl.pallas_call(
        _mid_tc,
        grid=(N_PAD // BN,),
        in_specs=[
            pl.BlockSpec((NC, BN, HALF), lambda i: (0, i, 0)),
            pl.BlockSpec((DIM_H, DIM_H), lambda i: (0, 0)),
            pl.BlockSpec((1, DIM_H), lambda i: (0, 0)),
            pl.BlockSpec((NW, BN), lambda i: (0, i)),
        ],
        out_specs=pl.BlockSpec((NC, BN, HALF), lambda i: (0, i, 0)),
        out_shape=jax.ShapeDtypeStruct((NC, N_PAD, HALF), jnp.float32),
    )(agg, w2, b1r, deg_parts)


def _out_call(agg, b2r, w3, b3r, deg_parts):
    return pl.pallas_call(
        _out_tc,
        grid=(N_PAD // BN,),
        in_specs=[
            pl.BlockSpec((NC, BN, HALF), lambda i: (0, i, 0)),
            pl.BlockSpec((1, DIM_H), lambda i: (0, 0)),
            pl.BlockSpec((DIM_H, DIM_OUT), lambda i: (0, 0)),
            pl.BlockSpec((1, DIM_OUT), lambda i: (0, 0)),
            pl.BlockSpec((NW, BN), lambda i: (0, i)),
        ],
        out_specs=pl.BlockSpec((BN, DIM_OUT), lambda i: (i, 0)),
        out_shape=jax.ShapeDtypeStruct((N_PAD, DIM_OUT), jnp.float32),
    )(agg, b2r, w3, b3r, deg_parts)


def kernel(x, edge_index, W1, b1, W2, b2, W3, b3):
    e = edge_index.shape[1]
    src = edge_index[0].astype(jnp.int32)
    dst = edge_index[1].astype(jnp.int32)

    eh = e // NW
    deg_parts = _make_hist(eh)(dst.reshape(NW, eh))

    em = -(-e // (NS * ESUP)) * ESUP
    pad = NS * em - e
    fill = jnp.full((pad,), N_NODES, jnp.int32)
    srcp = jnp.concatenate([src, fill]).reshape(NS, em // ESUP, KSUP, CB)
    dstp = jnp.concatenate([dst, fill]).reshape(NS, em // ESUP, KSUP, CB)
    mp = _make_mp(em // ESUP)

    xp = jnp.pad(x, ((0, N_PAD - N_NODES), (0, 0)))
    b1r = b1.reshape(1, DIM_H)
    b2r = b2.reshape(1, DIM_H)
    b3r = b3.reshape(1, DIM_OUT)

    hs1 = _lin1_call(xp, W1, deg_parts)
    agg1 = mp(hs1, srcp, dstp)
    hs2 = _mid_call(agg1, W2, b1r, deg_parts)
    agg2 = mp(hs2, srcp, dstp)
    return _out_call(agg2, b2r, W3, b3r, deg_parts)[:N_NODES]


# speedup vs baseline: 10.5753x; 1.0356x over previous
import functools

import jax
import jax.numpy as jnp
from jax import lax
from jax.experimental import pallas as pl
from jax.experimental.pallas import tpu as pltpu
from jax.experimental.pallas import tpu_sc as plsc

N_NODES = 10000
DIM_IN = 128
DIM_H = 256
DIM_OUT = 64

NC = 2
NS = 16
NW = NC * NS
L = 16

N_PAD = 10240
ROWS_PER_TILE = N_PAD // NS
HALF = DIM_H // 2
CB = 64


def _hist_body(eh, dst_hbm, out_hbm, dst_v, hist_v):
    c = lax.axis_index("c")
    s = lax.axis_index("s")
    wid = s * NC + c
    pltpu.sync_copy(dst_hbm.at[wid], dst_v)
    zeros16 = jnp.zeros((L,), jnp.float32)

    def zbody(g, carry):
        hist_v[pl.ds(g * L, L)] = zeros16
        return carry

    lax.fori_loop(0, N_PAD // L, zbody, 0)
    ones16 = jnp.ones((L,), jnp.float32)

    def body(g, carry):
        idx = dst_v[pl.ds(g * L, L)]
        plsc.addupdate_scatter(hist_v, [idx], ones16)
        return carry

    lax.fori_loop(0, eh // L, body, 0)
    pltpu.sync_copy(hist_v, out_hbm.at[wid])


def _make_hist(eh):
    return pl.kernel(
        functools.partial(_hist_body, eh),
        out_type=jax.ShapeDtypeStruct((NW, N_PAD), jnp.float32),
        mesh=plsc.VectorSubcoreMesh(core_axis_name="c", subcore_axis_name="s"),
        compiler_params=pltpu.CompilerParams(needs_layout_passes=False),
        scratch_types=[
            pltpu.VMEM((eh,), jnp.int32),
            pltpu.VMEM((N_PAD,), jnp.float32),
        ],
    )


KSUP = 32
ESUP = KSUP * CB
NBUF = 4
GAHEAD = 3


def _mp_body(nsup, hs_hbm, src_hbm, dst_hbm, out_hbm,
             src_buf, dst_buf, rows_v, agg_sh, *sems):
    gsem = sems[:NBUF]
    ssem = sems[NBUF:]
    c = lax.axis_index("c")
    s = lax.axis_index("s")
    r0 = s * ROWS_PER_TILE
    nblk = nsup * KSUP
    pltpu.sync_copy(hs_hbm.at[c, pl.ds(r0, ROWS_PER_TILE)],
                    agg_sh.at[pl.ds(r0, ROWS_PER_TILE)])
    plsc.subcore_barrier()
    hsc = hs_hbm.at[c]

    def idx_ref(buf, k):
        return buf.at[(k // KSUP) % 2, k % KSUP]

    def issue_gather(k, b):
        pltpu.async_copy(hsc.at[idx_ref(src_buf, k)], rows_v.at[b], gsem[b])

    def load_super(sup):
        pltpu.sync_copy(src_hbm.at[s, sup], src_buf.at[sup % 2])
        pltpu.sync_copy(dst_hbm.at[s, sup], dst_buf.at[sup % 2])

    load_super(0)
    for k0 in range(GAHEAD):
        issue_gather(k0, k0)

    def chunk(k, b):
        pltpu.make_async_copy(hsc.at[idx_ref(src_buf, k)], rows_v.at[b],
                              gsem[b]).wait()
        pltpu.async_copy(rows_v.at[b], agg_sh.at[idx_ref(dst_buf, k)],
                         ssem[b], add=True)
        kg = k + GAHEAD
        bg = (b + GAHEAD) % NBUF

        @pl.when(kg < nblk)
        def _():
            @pl.when((kg % KSUP == 0) & (kg // KSUP > 0))
            def _():
                load_super_dyn(kg // KSUP)

            @pl.when(kg >= NBUF)
            def _():
                pltpu.make_async_copy(rows_v.at[bg],
                                      agg_sh.at[idx_ref(dst_buf, kg - NBUF)],
                                      ssem[bg]).wait()

            issue_gather_dyn(k, kg, bg)

    def load_super_dyn(sup):
        pltpu.sync_copy(src_hbm.at[s, sup], src_buf.at[sup % 2])
        pltpu.sync_copy(dst_hbm.at[s, sup], dst_buf.at[sup % 2])

    def issue_gather_dyn(k, kg, bg):
        pltpu.async_copy(hsc.at[idx_ref(src_buf, kg)], rows_v.at[bg], gsem[bg])

    def group(p, carry):
        for b in range(NBUF):
            chunk(NBUF * p + b, b)
        return carry

    lax.fori_loop(0, nblk // NBUF, group, 0)
    for d in range(NBUF):
        k = nblk - NBUF + d
        pltpu.make_async_copy(rows_v.at[k % NBUF],
                              agg_sh.at[idx_ref(dst_buf, k)],
                              ssem[k % NBUF]).wait()
    plsc.subcore_barrier()
    pltpu.sync_copy(agg_sh.at[pl.ds(r0, ROWS_PER_TILE)],
                    out_hbm.at[c, pl.ds(r0, ROWS_PER_TILE)])


def _make_mp(nsup):
    return pl.kernel(
        functools.partial(_mp_body, nsup),
        out_type=jax.ShapeDtypeStruct((NC, N_PAD, HALF), jnp.float32),
        mesh=plsc.VectorSubcoreMesh(core_axis_name="c", subcore_axis_name="s"),
        compiler_params=pltpu.CompilerParams(needs_layout_passes=False),
        scratch_types=[
            pltpu.VMEM((2, KSUP, CB), jnp.int32),
            pltpu.VMEM((2, KSUP, CB), jnp.int32),
            pltpu.VMEM((NBUF, CB, HALF), jnp.float32),
            pltpu.VMEM_SHARED((N_PAD, HALF), jnp.float32),
        ] + [pltpu.SemaphoreType.DMA] * (2 * NBUF),
    )


def _dinv_from(deg_ref):
    dsum = jnp.sum(deg_ref[...], axis=0) + 1.0
    return lax.rsqrt(dsum)[:, None]


def _lin1_tc(x_ref, w_ref, deg_ref, out_ref):
    dinv = _dinv_from(deg_ref)
    h = jnp.dot(x_ref[...], w_ref[...], preferred_element_type=jnp.float32)
    hs = h * dinv
    out_ref[0] = hs[:, :HALF]
    out_ref[1] = hs[:, HALF:]


def _mid_tc(agg_ref, w_ref, b_ref, deg_ref, out_ref):
    dinv = _dinv_from(deg_ref)
    hl = jnp.maximum(agg_ref[0] * dinv + b_ref[:, :HALF], 0.0)
    hr = jnp.maximum(agg_ref[1] * dinv + b_ref[:, HALF:], 0.0)
    h2 = (jnp.dot(hl, w_ref[:HALF, :], preferred_element_type=jnp.float32)
          + jnp.dot(hr, w_ref[HALF:, :], preferred_element_type=jnp.float32))
    hs = h2 * dinv
    out_ref[0] = hs[:, :HALF]
    out_ref[1] = hs[:, HALF:]


def _out_tc(agg_ref, b2_ref, w3_ref, b3_ref, deg_ref, out_ref):
    dinv = _dinv_from(deg_ref)
    hl = jnp.maximum(agg_ref[0] * dinv + b2_ref[:, :HALF], 0.0)
    hr = jnp.maximum(agg_ref[1] * dinv + b2_ref[:, HALF:], 0.0)
    logits = (jnp.dot(hl, w3_ref[:HALF, :], preferred_element_type=jnp.float32)
              + jnp.dot(hr, w3_ref[HALF:, :], preferred_element_type=jnp.float32)
              + b3_ref[...])
    m = jnp.max(logits, axis=1, keepdims=True)
    sh = logits - m
    lse = jnp.log(jnp.sum(jnp.exp(sh), axis=1, keepdims=True))
    out_ref[...] = sh - lse


BN = 1024


def _lin1_call(xp, w1, deg_parts):
    return pl.pallas_call(
        _lin1_tc,
        grid=(N_PAD // BN,),
        in_specs=[
            pl.BlockSpec((BN, DIM_IN), lambda i: (i, 0)),
            pl.BlockSpec((DIM_IN, DIM_H), lambda i: (0, 0)),
            pl.BlockSpec((NW, BN), lambda i: (0, i)),
        ],
        out_specs=pl.BlockSpec((NC, BN, HALF), lambda i: (0, i, 0)),
        out_shape=jax.ShapeDtypeStruct((NC, N_PAD, HALF), jnp.float32),
    )(xp, w1, deg_parts)


def _mid_call(agg, w2, b1r, deg_parts):
    return pl.pallas_call(
        _mid_tc,
        grid=(N_PAD // BN,),
        in_specs=[
            pl.BlockSpec((NC, BN, HALF), lambda i: (0, i, 0)),
            pl.BlockSpec((DIM_H, DIM_H), lambda i: (0, 0)),
            pl.BlockSpec((1, DIM_H), lambda i: (0, 0)),
            pl.BlockSpec((NW, BN), lambda i: (0, i)),
        ],
        out_specs=pl.BlockSpec((NC, BN, HALF), lambda i: (0, i, 0)),
        out_shape=jax.ShapeDtypeStruct((NC, N_PAD, HALF), jnp.float32),
    )(agg, w2, b1r, deg_parts)


def _out_call(agg, b2r, w3, b3r, deg_parts):
    return pl.pallas_call(
        _out_tc,
        grid=(N_PAD // BN,),
        in_specs=[
            pl.BlockSpec((NC, BN, HALF), lambda i: (0, i, 0)),
            pl.BlockSpec((1, DIM_H), lambda i: (0, 0)),
            pl.BlockSpec((DIM_H, DIM_OUT), lambda i: (0, 0)),
            pl.BlockSpec((1, DIM_OUT), lambda i: (0, 0)),
            pl.BlockSpec((NW, BN), lambda i: (0, i)),
        ],
        out_specs=pl.BlockSpec((BN, DIM_OUT), lambda i: (i, 0)),
        out_shape=jax.ShapeDtypeStruct((N_PAD, DIM_OUT), jnp.float32),
    )(agg, b2r, w3, b3r, deg_parts)


def kernel(x, edge_index, W1, b1, W2, b2, W3, b3):
    e = edge_index.shape[1]
    src = edge_index[0].astype(jnp.int32)
    dst = edge_index[1].astype(jnp.int32)

    eh = e // NW
    deg_parts = _make_hist(eh)(dst.reshape(NW, eh))

    em = -(-e // (NS * ESUP)) * ESUP
    pad = NS * em - e
    fill = jnp.full((pad,), N_NODES, jnp.int32)
    srcp = jnp.concatenate([src, fill]).reshape(NS, em // ESUP, KSUP, CB)
    dstp = jnp.concatenate([dst, fill]).reshape(NS, em // ESUP, KSUP, CB)
    mp = _make_mp(em // ESUP)

    xp = jnp.pad(x, ((0, N_PAD - N_NODES), (0, 0)))
    b1r = b1.reshape(1, DIM_H)
    b2r = b2.reshape(1, DIM_H)
    b3r = b3.reshape(1, DIM_OUT)

    hs1 = _lin1_call(xp, W1, deg_parts)
    agg1 = mp(hs1, srcp, dstp)
    hs2 = _mid_call(agg1, W2, b1r, deg_parts)
    agg2 = mp(hs2, srcp, dstp)
    return _out_call(agg2, b2r, W3, b3r, deg_parts)[:N_NODES]


# speedup vs baseline: 18.6791x; 1.7663x over previous
import functools

import jax
import jax.numpy as jnp
from jax import lax
from jax.experimental import pallas as pl
from jax.experimental.pallas import tpu as pltpu
from jax.experimental.pallas import tpu_sc as plsc

N_NODES = 10000
DIM_IN = 128
DIM_H = 256
DIM_OUT = 64

NC = 2
NS = 16
NW = NC * NS
L = 16

N_PAD = 10240
ROWS_PER_TILE = N_PAD // NS
NQ = 4
QC = DIM_H // NQ
CB = 128


def _hist_body(eh, dst_hbm, out_hbm, dst_v, hist_v):
    c = lax.axis_index("c")
    s = lax.axis_index("s")
    wid = s * NC + c
    pltpu.sync_copy(dst_hbm.at[wid], dst_v)
    zeros16 = jnp.zeros((L,), jnp.float32)

    def zbody(g, carry):
        hist_v[pl.ds(g * L, L)] = zeros16
        return carry

    lax.fori_loop(0, N_PAD // L, zbody, 0)
    ones16 = jnp.ones((L,), jnp.float32)

    def body(g, carry):
        idx = dst_v[pl.ds(g * L, L)]
        plsc.addupdate_scatter(hist_v, [idx], ones16)
        return carry

    lax.fori_loop(0, eh // L, body, 0)
    pltpu.sync_copy(hist_v, out_hbm.at[wid])


def _make_hist(eh):
    return pl.kernel(
        functools.partial(_hist_body, eh),
        out_type=jax.ShapeDtypeStruct((NW, N_PAD), jnp.float32),
        mesh=plsc.VectorSubcoreMesh(core_axis_name="c", subcore_axis_name="s"),
        compiler_params=pltpu.CompilerParams(needs_layout_passes=False),
        scratch_types=[
            pltpu.VMEM((eh,), jnp.int32),
            pltpu.VMEM((N_PAD,), jnp.float32),
        ],
    )


KSUP = 16
ESUP = KSUP * CB
NBUF = 4
GAHEAD = 2


def _mp_body(nsup, hs_hbm, src_hbm, dst_hbm, out_hbm,
             src_buf, dst_buf, rows_v, hs_sp, agg_sp, *sems):
    gsem = sems[:NBUF]
    ssem = sems[NBUF:]
    c = lax.axis_index("c")
    s = lax.axis_index("s")
    r0 = s * ROWS_PER_TILE
    nblk = nsup * KSUP

    def idx_ref(buf, k):
        return buf.at[(k // KSUP) % 2, k % KSUP]

    def issue_gather(kg, bg):
        pltpu.async_copy(hs_sp.at[idx_ref(src_buf, kg)], rows_v.at[bg],
                         gsem[bg])

    for p in range(NQ // NC):
        q = c * (NQ // NC) + p
        pltpu.sync_copy(hs_hbm.at[q, pl.ds(r0, ROWS_PER_TILE)],
                        hs_sp.at[pl.ds(r0, ROWS_PER_TILE)])
        pltpu.sync_copy(hs_hbm.at[q, pl.ds(r0, ROWS_PER_TILE)],
                        agg_sp.at[pl.ds(r0, ROWS_PER_TILE)])
        pltpu.sync_copy(src_hbm.at[s, 0], src_buf.at[0])
        pltpu.sync_copy(dst_hbm.at[s, 0], dst_buf.at[0])
        plsc.subcore_barrier()
        for k0 in range(GAHEAD):
            issue_gather(k0, k0)

        def chunk(k, b):
            pltpu.make_async_copy(hs_sp.at[idx_ref(src_buf, k)],
                                  rows_v.at[b], gsem[b]).wait()
            pltpu.async_copy(rows_v.at[b], agg_sp.at[idx_ref(dst_buf, k)],
                             ssem[b], add=True)
            kg = k + GAHEAD
            bg = (b + GAHEAD) % NBUF

            @pl.when(kg < nblk)
            def _():
                @pl.when((kg % KSUP == 0) & (kg // KSUP > 0))
                def _():
                    pltpu.sync_copy(src_hbm.at[s, kg // KSUP],
                                    src_buf.at[(kg // KSUP) % 2])
                    pltpu.sync_copy(dst_hbm.at[s, kg // KSUP],
                                    dst_buf.at[(kg // KSUP) % 2])

                @pl.when(kg >= NBUF)
                def _():
                    pltpu.make_async_copy(
                        rows_v.at[bg],
                        agg_sp.at[idx_ref(dst_buf, kg - NBUF)],
                        ssem[bg]).wait()

                issue_gather(kg, bg)

        def group(g, carry):
            for b in range(NBUF):
                chunk(NBUF * g + b, b)
            return carry

        lax.fori_loop(0, nblk // NBUF, group, 0)
        for d in range(NBUF):
            k = nblk - NBUF + d
            pltpu.make_async_copy(rows_v.at[k % NBUF],
                                  agg_sp.at[idx_ref(dst_buf, k)],
                                  ssem[k % NBUF]).wait()
        plsc.subcore_barrier()
        pltpu.sync_copy(agg_sp.at[pl.ds(r0, ROWS_PER_TILE)],
                        out_hbm.at[q, pl.ds(r0, ROWS_PER_TILE)])


def _make_mp(nsup):
    return pl.kernel(
        functools.partial(_mp_body, nsup),
        out_type=jax.ShapeDtypeStruct((NQ, N_PAD, QC), jnp.float32),
        mesh=plsc.VectorSubcoreMesh(core_axis_name="c", subcore_axis_name="s"),
        compiler_params=pltpu.CompilerParams(needs_layout_passes=False,
                                             use_tc_tiling_on_sc=False),
        scratch_types=[
            pltpu.VMEM((2, KSUP, CB), jnp.int32),
            pltpu.VMEM((2, KSUP, CB), jnp.int32),
            pltpu.VMEM((NBUF, CB, QC), jnp.float32),
            pltpu.VMEM_SHARED((N_PAD, QC), jnp.float32),
            pltpu.VMEM_SHARED((N_PAD, QC), jnp.float32),
        ] + [pltpu.SemaphoreType.DMA] * (2 * NBUF),
    )


def _dinv_from(deg_ref):
    dsum = jnp.sum(deg_ref[...], axis=0) + 1.0
    return lax.rsqrt(dsum)[:, None]


def _store_quarters(out_ref, hs):
    for i in range(NQ):
        out_ref[i] = hs[:, i * QC:(i + 1) * QC]


def _lin1_tc(x_ref, w_ref, deg_ref, out_ref):
    dinv = _dinv_from(deg_ref)
    h = jnp.dot(x_ref[...], w_ref[...], preferred_element_type=jnp.float32)
    _store_quarters(out_ref, h * dinv)


def _relu_quarters(agg_ref, b_ref, dinv):
    return [jnp.maximum(agg_ref[i] * dinv + b_ref[:, i * QC:(i + 1) * QC], 0.0)
            for i in range(NQ)]


def _mid_tc(agg_ref, w_ref, b_ref, deg_ref, out_ref):
    dinv = _dinv_from(deg_ref)
    hq = _relu_quarters(agg_ref, b_ref, dinv)
    h2 = sum(jnp.dot(hq[i], w_ref[i * QC:(i + 1) * QC, :],
                     preferred_element_type=jnp.float32) for i in range(NQ))
    _store_quarters(out_ref, h2 * dinv)


def _out_tc(agg_ref, b2_ref, w3_ref, b3_ref, deg_ref, out_ref):
    dinv = _dinv_from(deg_ref)
    hq = _relu_quarters(agg_ref, b2_ref, dinv)
    logits = sum(jnp.dot(hq[i], w3_ref[i * QC:(i + 1) * QC, :],
                         preferred_element_type=jnp.float32)
                 for i in range(NQ)) + b3_ref[...]
    m = jnp.max(logits, axis=1, keepdims=True)
    sh = logits - m
    lse = jnp.log(jnp.sum(jnp.exp(sh), axis=1, keepdims=True))
    out_ref[...] = sh - lse


BN = 1024


def _lin1_call(xp, w1, deg_parts):
    return pl.pallas_call(
        _lin1_tc,
        grid=(N_PAD // BN,),
        in_specs=[
            pl.BlockSpec((BN, DIM_IN), lambda i: (i, 0)),
            pl.BlockSpec((DIM_IN, DIM_H), lambda i: (0, 0)),
            pl.BlockSpec((NW, BN), lambda i: (0, i)),
        ],
        out_specs=pl.BlockSpec((NQ, BN, QC), lambda i: (0, i, 0)),
        out_shape=jax.ShapeDtypeStruct((NQ, N_PAD, QC), jnp.float32),
    )(xp, w1, deg_parts)


def _mid_call(agg, w2, b1r, deg_parts):
    return pl.pallas_call(
        _mid_tc,
        grid=(N_PAD // BN,),
        in_specs=[
            pl.BlockSpec((NQ, BN, QC), lambda i: (0, i, 0)),
            pl.BlockSpec((DIM_H, DIM_H), lambda i: (0, 0)),
            pl.BlockSpec((1, DIM_H), lambda i: (0, 0)),
            pl.BlockSpec((NW, BN), lambda i: (0, i)),
        ],
        out_specs=pl.BlockSpec((NQ, BN, QC), lambda i: (0, i, 0)),
        out_shape=jax.ShapeDtypeStruct((NQ, N_PAD, QC), jnp.float32),
    )(agg, w2, b1r, deg_parts)


def _out_call(agg, b2r, w3, b3r, deg_parts):
    return pl.pallas_call(
        _out_tc,
        grid=(N_PAD // BN,),
        in_specs=[
            pl.BlockSpec((NQ, BN, QC), lambda i: (0, i, 0)),
            pl.BlockSpec((1, DIM_H), lambda i: (0, 0)),
            pl.BlockSpec((DIM_H, DIM_OUT), lambda i: (0, 0)),
            pl.BlockSpec((1, DIM_OUT), lambda i: (0, 0)),
            pl.BlockSpec((NW, BN), lambda i: (0, i)),
        ],
        out_specs=pl.BlockSpec((BN, DIM_OUT), lambda i: (i, 0)),
        out_shape=jax.ShapeDtypeStruct((N_PAD, DIM_OUT), jnp.float32),
    )(agg, b2r, w3, b3r, deg_parts)


def kernel(x, edge_index, W1, b1, W2, b2, W3, b3):
    e = edge_index.shape[1]
    src = edge_index[0].astype(jnp.int32)
    dst = edge_index[1].astype(jnp.int32)

    eh = e // NW
    deg_parts = _make_hist(eh)(dst.reshape(NW, eh))

    em = -(-e // (NS * ESUP)) * ESUP
    pad = NS * em - e
    fill = jnp.full((pad,), N_NODES, jnp.int32)
    srcp = jnp.concatenate([src, fill]).reshape(NS, em // ESUP, KSUP, CB)
    dstp = jnp.concatenate([dst, fill]).reshape(NS, em // ESUP, KSUP, CB)
    mp = _make_mp(em // ESUP)

    xp = jnp.pad(x, ((0, N_PAD - N_NODES), (0, 0)))
    b1r = b1.reshape(1, DIM_H)
    b2r = b2.reshape(1, DIM_H)
    b3r = b3.reshape(1, DIM_OUT)

    hs1 = _lin1_call(xp, W1, deg_parts)
    agg1 = mp(hs1, srcp, dstp)
    hs2 = _mid_call(agg1, W2, b1r, deg_parts)
    agg2 = mp(hs2, srcp, dstp)
    return _out_call(agg2, b2r, W3, b3r, deg_parts)[:N_NODES]


# speedup vs baseline: 19.0538x; 1.0201x over previous
import functools

import jax
import jax.numpy as jnp
from jax import lax
from jax.experimental import pallas as pl
from jax.experimental.pallas import tpu as pltpu
from jax.experimental.pallas import tpu_sc as plsc

N_NODES = 10000
DIM_IN = 128
DIM_H = 256
DIM_OUT = 64

NC = 2
NS = 16
NW = NC * NS
L = 16

N_PAD = 10240
ROWS_PER_TILE = N_PAD // NS
NQ = 4
QC = DIM_H // NQ
CB = 128


def _hist_body(eh, dst_hbm, out_hbm, dst_v, hist_v):
    c = lax.axis_index("c")
    s = lax.axis_index("s")
    wid = s * NC + c
    pltpu.sync_copy(dst_hbm.at[wid], dst_v)
    zeros16 = jnp.zeros((L,), jnp.float32)

    def zbody(g, carry):
        hist_v[pl.ds(g * L, L)] = zeros16
        return carry

    lax.fori_loop(0, N_PAD // L, zbody, 0)
    ones16 = jnp.ones((L,), jnp.float32)

    def body(g, carry):
        idx = dst_v[pl.ds(g * L, L)]
        plsc.addupdate_scatter(hist_v, [idx], ones16)
        return carry

    lax.fori_loop(0, eh // L, body, 0)
    pltpu.sync_copy(hist_v, out_hbm.at[wid])


def _make_hist(eh):
    return pl.kernel(
        functools.partial(_hist_body, eh),
        out_type=jax.ShapeDtypeStruct((NW, N_PAD), jnp.float32),
        mesh=plsc.VectorSubcoreMesh(core_axis_name="c", subcore_axis_name="s"),
        compiler_params=pltpu.CompilerParams(needs_layout_passes=False),
        scratch_types=[
            pltpu.VMEM((eh,), jnp.int32),
            pltpu.VMEM((N_PAD,), jnp.float32),
        ],
    )


KSUP = 16
ESUP = KSUP * CB
NBUF = 4
GAHEAD = 3


def _mp_body(nsup, hs_hbm, src_hbm, dst_hbm, out_hbm,
             src_buf, dst_buf, rows_v, hs_sp, agg_sp, *sems):
    gsem = sems[:NBUF]
    ssem = sems[NBUF:]
    c = lax.axis_index("c")
    s = lax.axis_index("s")
    r0 = s * ROWS_PER_TILE
    nblk = nsup * KSUP

    def idx_ref(buf, k):
        return buf.at[(k // KSUP) % 2, k % KSUP]

    def issue_gather(kg, bg):
        pltpu.async_copy(hs_sp.at[idx_ref(src_buf, kg)], rows_v.at[bg],
                         gsem[bg])

    for p in range(NQ // NC):
        q = c * (NQ // NC) + p
        pltpu.sync_copy(hs_hbm.at[q, pl.ds(r0, ROWS_PER_TILE)],
                        hs_sp.at[pl.ds(r0, ROWS_PER_TILE)])
        pltpu.sync_copy(hs_hbm.at[q, pl.ds(r0, ROWS_PER_TILE)],
                        agg_sp.at[pl.ds(r0, ROWS_PER_TILE)])
        pltpu.sync_copy(src_hbm.at[s, 0], src_buf.at[0])
        pltpu.sync_copy(dst_hbm.at[s, 0], dst_buf.at[0])
        plsc.subcore_barrier()
        for k0 in range(GAHEAD):
            issue_gather(k0, k0)

        def chunk(k, b):
            pltpu.make_async_copy(hs_sp.at[idx_ref(src_buf, k)],
                                  rows_v.at[b], gsem[b]).wait()
            pltpu.async_copy(rows_v.at[b], agg_sp.at[idx_ref(dst_buf, k)],
                             ssem[b], add=True)
            kg = k + GAHEAD
            bg = (b + GAHEAD) % NBUF

            @pl.when(kg < nblk)
            def _():
                @pl.when((kg % KSUP == 0) & (kg // KSUP > 0))
                def _():
                    pltpu.sync_copy(src_hbm.at[s, kg // KSUP],
                                    src_buf.at[(kg // KSUP) % 2])
                    pltpu.sync_copy(dst_hbm.at[s, kg // KSUP],
                                    dst_buf.at[(kg // KSUP) % 2])

                @pl.when(kg >= NBUF)
                def _():
                    pltpu.make_async_copy(
                        rows_v.at[bg],
                        agg_sp.at[idx_ref(dst_buf, kg - NBUF)],
                        ssem[bg]).wait()

                issue_gather(kg, bg)

        def group(g, carry):
            for b in range(NBUF):
                chunk(NBUF * g + b, b)
            return carry

        lax.fori_loop(0, nblk // NBUF, group, 0)
        for d in range(NBUF):
            k = nblk - NBUF + d
            pltpu.make_async_copy(rows_v.at[k % NBUF],
                                  agg_sp.at[idx_ref(dst_buf, k)],
                                  ssem[k % NBUF]).wait()
        plsc.subcore_barrier()
        pltpu.sync_copy(agg_sp.at[pl.ds(r0, ROWS_PER_TILE)],
                        out_hbm.at[q, pl.ds(r0, ROWS_PER_TILE)])


def _make_mp(nsup):
    return pl.kernel(
        functools.partial(_mp_body, nsup),
        out_type=jax.ShapeDtypeStruct((NQ, N_PAD, QC), jnp.float32),
        mesh=plsc.VectorSubcoreMesh(core_axis_name="c", subcore_axis_name="s"),
        compiler_params=pltpu.CompilerParams(needs_layout_passes=False,
                                             use_tc_tiling_on_sc=False),
        scratch_types=[
            pltpu.VMEM((2, KSUP, CB), jnp.int32),
            pltpu.VMEM((2, KSUP, CB), jnp.int32),
            pltpu.VMEM((NBUF, CB, QC), jnp.float32),
            pltpu.VMEM_SHARED((N_PAD, QC), jnp.float32),
            pltpu.VMEM_SHARED((N_PAD, QC), jnp.float32),
        ] + [pltpu.SemaphoreType.DMA] * (2 * NBUF),
    )


def _dinv_from(deg_ref):
    dsum = jnp.sum(deg_ref[...], axis=0) + 1.0
    return lax.rsqrt(dsum)[:, None]


def _store_quarters(out_ref, hs):
    for i in range(NQ):
        out_ref[i] = hs[:, i * QC:(i + 1) * QC]


def _lin1_tc(x_ref, w_ref, deg_ref, out_ref):
    dinv = _dinv_from(deg_ref)
    h = jnp.dot(x_ref[...], w_ref[...], preferred_element_type=jnp.float32)
    _store_quarters(out_ref, h * dinv)


def _relu_quarters(agg_ref, b_ref, dinv):
    return [jnp.maximum(agg_ref[i] * dinv + b_ref[:, i * QC:(i + 1) * QC], 0.0)
            for i in range(NQ)]


def _mid_tc(agg_ref, w_ref, b_ref, deg_ref, out_ref):
    dinv = _dinv_from(deg_ref)
    hq = _relu_quarters(agg_ref, b_ref, dinv)
    h2 = sum(jnp.dot(hq[i], w_ref[i * QC:(i + 1) * QC, :],
                     preferred_element_type=jnp.float32) for i in range(NQ))
    _store_quarters(out_ref, h2 * dinv)


def _out_tc(agg_ref, b2_ref, w3_ref, b3_ref, deg_ref, out_ref):
    dinv = _dinv_from(deg_ref)
    hq = _relu_quarters(agg_ref, b2_ref, dinv)
    logits = sum(jnp.dot(hq[i], w3_ref[i * QC:(i + 1) * QC, :],
                         preferred_element_type=jnp.float32)
                 for i in range(NQ)) + b3_ref[...]
    m = jnp.max(logits, axis=1, keepdims=True)
    sh = logits - m
    lse = jnp.log(jnp.sum(jnp.exp(sh), axis=1, keepdims=True))
    out_ref[...] = sh - lse


BN = 1024


def _lin1_call(xp, w1, deg_parts):
    return pl.pallas_call(
        _lin1_tc,
        grid=(N_PAD // BN,),
        in_specs=[
            pl.BlockSpec((BN, DIM_IN), lambda i: (i, 0)),
            pl.BlockSpec((DIM_IN, DIM_H), lambda i: (0, 0)),
            pl.BlockSpec((NW, BN), lambda i: (0, i)),
        ],
        out_specs=pl.BlockSpec((NQ, BN, QC), lambda i: (0, i, 0)),
        out_shape=jax.ShapeDtypeStruct((NQ, N_PAD, QC), jnp.float32),
    )(xp, w1, deg_parts)


def _mid_call(agg, w2, b1r, deg_parts):
    return pl.pallas_call(
        _mid_tc,
        grid=(N_PAD // BN,),
        in_specs=[
            pl.BlockSpec((NQ, BN, QC), lambda i: (0, i, 0)),
            pl.BlockSpec((DIM_H, DIM_H), lambda i: (0, 0)),
            pl.BlockSpec((1, DIM_H), lambda i: (0, 0)),
            pl.BlockSpec((NW, BN), lambda i: (0, i)),
        ],
        out_specs=pl.BlockSpec((NQ, BN, QC), lambda i: (0, i, 0)),
        out_shape=jax.ShapeDtypeStruct((NQ, N_PAD, QC), jnp.float32),
    )(agg, w2, b1r, deg_parts)


def _out_call(agg, b2r, w3, b3r, deg_parts):
    return pl.pallas_call(
        _out_tc,
        grid=(N_PAD // BN,),
        in_specs=[
            pl.BlockSpec((NQ, BN, QC), lambda i: (0, i, 0)),
            pl.BlockSpec((1, DIM_H), lambda i: (0, 0)),
            pl.BlockSpec((DIM_H, DIM_OUT), lambda i: (0, 0)),
            pl.BlockSpec((1, DIM_OUT), lambda i: (0, 0)),
            pl.BlockSpec((NW, BN), lambda i: (0, i)),
        ],
        out_specs=pl.BlockSpec((BN, DIM_OUT), lambda i: (i, 0)),
        out_shape=jax.ShapeDtypeStruct((N_PAD, DIM_OUT), jnp.float32),
    )(agg, b2r, w3, b3r, deg_parts)


def kernel(x, edge_index, W1, b1, W2, b2, W3, b3):
    e = edge_index.shape[1]
    src = edge_index[0].astype(jnp.int32)
    dst = edge_index[1].astype(jnp.int32)

    eh = e // NW
    deg_parts = _make_hist(eh)(dst.reshape(NW, eh))

    em = -(-e // (NS * ESUP)) * ESUP
    pad = NS * em - e
    fill = jnp.full((pad,), N_NODES, jnp.int32)
    srcp = jnp.concatenate([src, fill]).reshape(NS, em // ESUP, KSUP, CB)
    dstp = jnp.concatenate([dst, fill]).reshape(NS, em // ESUP, KSUP, CB)
    mp = _make_mp(em // ESUP)

    xp = jnp.pad(x, ((0, N_PAD - N_NODES), (0, 0)))
    b1r = b1.reshape(1, DIM_H)
    b2r = b2.reshape(1, DIM_H)
    b3r = b3.reshape(1, DIM_OUT)

    hs1 = _lin1_call(xp, W1, deg_parts)
    agg1 = mp(hs1, srcp, dstp)
    hs2 = _mid_call(agg1, W2, b1r, deg_parts)
    agg2 = mp(hs2, srcp, dstp)
    return _out_call(agg2, b2r, W3, b3r, deg_parts)[:N_NODES]


# speedup vs baseline: 19.2499x; 1.0103x over previous
import functools

import jax
import jax.numpy as jnp
from jax import lax
from jax.experimental import pallas as pl
from jax.experimental.pallas import tpu as pltpu
from jax.experimental.pallas import tpu_sc as plsc

N_NODES = 10000
DIM_IN = 128
DIM_H = 256
DIM_OUT = 64

NC = 2
NS = 16
NW = NC * NS
L = 16

N_PAD = 10240
ROWS_PER_TILE = N_PAD // NS
NQ = 4
QC = DIM_H // NQ
CB = 128


def _hist_body(eh, dst_hbm, out_hbm, dst_v, hist_v):
    c = lax.axis_index("c")
    s = lax.axis_index("s")
    wid = s * NC + c
    pltpu.sync_copy(dst_hbm.at[wid], dst_v)
    zeros16 = jnp.zeros((L,), jnp.float32)

    def zbody(g, carry):
        hist_v[pl.ds(g * L, L)] = zeros16
        return carry

    lax.fori_loop(0, N_PAD // L, zbody, 0)
    ones16 = jnp.ones((L,), jnp.float32)

    def body(g, carry):
        idx = dst_v[pl.ds(g * L, L)]
        plsc.addupdate_scatter(hist_v, [idx], ones16)
        return carry

    lax.fori_loop(0, eh // L, body, 0)
    pltpu.sync_copy(hist_v, out_hbm.at[wid])


def _make_hist(eh):
    return pl.kernel(
        functools.partial(_hist_body, eh),
        out_type=jax.ShapeDtypeStruct((NW, N_PAD), jnp.float32),
        mesh=plsc.VectorSubcoreMesh(core_axis_name="c", subcore_axis_name="s"),
        compiler_params=pltpu.CompilerParams(needs_layout_passes=False),
        scratch_types=[
            pltpu.VMEM((eh,), jnp.int32),
            pltpu.VMEM((N_PAD,), jnp.float32),
        ],
    )


KSUP = 16
ESUP = KSUP * CB
NBUF = 4
GAHEAD = 3


def _mp_body(nsup, hs_hbm, src_hbm, dst_hbm, out_hbm,
             src_buf, dst_buf, rows_v, hs_sp, agg_sp, *sems):
    gsem = sems[:NBUF]
    ssem = sems[NBUF:]
    c = lax.axis_index("c")
    s = lax.axis_index("s")
    r0 = s * ROWS_PER_TILE
    nblk = nsup * KSUP

    def idx_ref(buf, k):
        return buf.at[(k // KSUP) % 2, k % KSUP]

    def issue_gather(kg, bg):
        pltpu.async_copy(hs_sp.at[idx_ref(src_buf, kg)], rows_v.at[bg],
                         gsem[bg])

    for p in range(NQ // NC):
        q = c * (NQ // NC) + p
        seeds = [
            (hs_hbm.at[q, pl.ds(r0, ROWS_PER_TILE)],
             hs_sp.at[pl.ds(r0, ROWS_PER_TILE)], gsem[0]),
            (hs_hbm.at[q, pl.ds(r0, ROWS_PER_TILE)],
             agg_sp.at[pl.ds(r0, ROWS_PER_TILE)], gsem[1]),
            (src_hbm.at[s, 0], src_buf.at[0], ssem[0]),
            (dst_hbm.at[s, 0], dst_buf.at[0], ssem[1]),
        ]
        for sref, dref, sem in seeds:
            pltpu.async_copy(sref, dref, sem)
        for sref, dref, sem in seeds:
            pltpu.make_async_copy(sref, dref, sem).wait()
        plsc.subcore_barrier()
        for k0 in range(GAHEAD):
            issue_gather(k0, k0)

        def chunk(k, b):
            pltpu.make_async_copy(hs_sp.at[idx_ref(src_buf, k)],
                                  rows_v.at[b], gsem[b]).wait()
            pltpu.async_copy(rows_v.at[b], agg_sp.at[idx_ref(dst_buf, k)],
                             ssem[b], add=True)
            kg = k + GAHEAD
            bg = (b + GAHEAD) % NBUF

            @pl.when(kg < nblk)
            def _():
                @pl.when((kg % KSUP == 0) & (kg // KSUP > 0))
                def _():
                    pltpu.sync_copy(src_hbm.at[s, kg // KSUP],
                                    src_buf.at[(kg // KSUP) % 2])
                    pltpu.sync_copy(dst_hbm.at[s, kg // KSUP],
                                    dst_buf.at[(kg // KSUP) % 2])

                @pl.when(kg >= NBUF)
                def _():
                    pltpu.make_async_copy(
                        rows_v.at[bg],
                        agg_sp.at[idx_ref(dst_buf, kg - NBUF)],
                        ssem[bg]).wait()

                issue_gather(kg, bg)

        def group(g, carry):
            for b in range(NBUF):
                chunk(NBUF * g + b, b)
            return carry

        lax.fori_loop(0, nblk // NBUF, group, 0)
        for d in range(NBUF):
            k = nblk - NBUF + d
            pltpu.make_async_copy(rows_v.at[k % NBUF],
                                  agg_sp.at[idx_ref(dst_buf, k)],
                                  ssem[k % NBUF]).wait()
        plsc.subcore_barrier()
        pltpu.sync_copy(agg_sp.at[pl.ds(r0, ROWS_PER_TILE)],
                        out_hbm.at[q, pl.ds(r0, ROWS_PER_TILE)])


def _make_mp(nsup):
    return pl.kernel(
        functools.partial(_mp_body, nsup),
        out_type=jax.ShapeDtypeStruct((NQ, N_PAD, QC), jnp.float32),
        mesh=plsc.VectorSubcoreMesh(core_axis_name="c", subcore_axis_name="s"),
        compiler_params=pltpu.CompilerParams(needs_layout_passes=False,
                                             use_tc_tiling_on_sc=False),
        scratch_types=[
            pltpu.VMEM((2, KSUP, CB), jnp.int32),
            pltpu.VMEM((2, KSUP, CB), jnp.int32),
            pltpu.VMEM((NBUF, CB, QC), jnp.float32),
            pltpu.VMEM_SHARED((N_PAD, QC), jnp.float32),
            pltpu.VMEM_SHARED((N_PAD, QC), jnp.float32),
        ] + [pltpu.SemaphoreType.DMA] * (2 * NBUF),
    )


def _dinv_from(deg_ref):
    dsum = jnp.sum(deg_ref[...], axis=0) + 1.0
    return lax.rsqrt(dsum)[:, None]


def _store_quarters(out_ref, hs):
    for i in range(NQ):
        out_ref[i] = hs[:, i * QC:(i + 1) * QC]


def _lin1_tc(x_ref, w_ref, deg_ref, out_ref):
    dinv = _dinv_from(deg_ref)
    h = jnp.dot(x_ref[...], w_ref[...], preferred_element_type=jnp.float32)
    _store_quarters(out_ref, h * dinv)


def _relu_quarters(agg_ref, b_ref, dinv):
    return [jnp.maximum(agg_ref[i] * dinv + b_ref[:, i * QC:(i + 1) * QC], 0.0)
            for i in range(NQ)]


def _mid_tc(agg_ref, w_ref, b_ref, deg_ref, out_ref):
    dinv = _dinv_from(deg_ref)
    hq = _relu_quarters(agg_ref, b_ref, dinv)
    h2 = sum(jnp.dot(hq[i], w_ref[i * QC:(i + 1) * QC, :],
                     preferred_element_type=jnp.float32) for i in range(NQ))
    _store_quarters(out_ref, h2 * dinv)


def _out_tc(agg_ref, b2_ref, w3_ref, b3_ref, deg_ref, out_ref):
    dinv = _dinv_from(deg_ref)
    hq = _relu_quarters(agg_ref, b2_ref, dinv)
    logits = sum(jnp.dot(hq[i], w3_ref[i * QC:(i + 1) * QC, :],
                         preferred_element_type=jnp.float32)
                 for i in range(NQ)) + b3_ref[...]
    m = jnp.max(logits, axis=1, keepdims=True)
    sh = logits - m
    lse = jnp.log(jnp.sum(jnp.exp(sh), axis=1, keepdims=True))
    out_ref[...] = sh - lse


BN = 1024


def _lin1_call(xp, w1, deg_parts):
    return pl.pallas_call(
        _lin1_tc,
        grid=(N_PAD // BN,),
        in_specs=[
            pl.BlockSpec((BN, DIM_IN), lambda i: (i, 0)),
            pl.BlockSpec((DIM_IN, DIM_H), lambda i: (0, 0)),
            pl.BlockSpec((NW, BN), lambda i: (0, i)),
        ],
        out_specs=pl.BlockSpec((NQ, BN, QC), lambda i: (0, i, 0)),
        out_shape=jax.ShapeDtypeStruct((NQ, N_PAD, QC), jnp.float32),
    )(xp, w1, deg_parts)


def _mid_call(agg, w2, b1r, deg_parts):
    return pl.pallas_call(
        _mid_tc,
        grid=(N_PAD // BN,),
        in_specs=[
            pl.BlockSpec((NQ, BN, QC), lambda i: (0, i, 0)),
            pl.BlockSpec((DIM_H, DIM_H), lambda i: (0, 0)),
            pl.BlockSpec((1, DIM_H), lambda i: (0, 0)),
            pl.BlockSpec((NW, BN), lambda i: (0, i)),
        ],
        out_specs=pl.BlockSpec((NQ, BN, QC), lambda i: (0, i, 0)),
        out_shape=jax.ShapeDtypeStruct((NQ, N_PAD, QC), jnp.float32),
    )(agg, w2, b1r, deg_parts)


def _out_call(agg, b2r, w3, b3r, deg_parts):
    return pl.pallas_call(
        _out_tc,
        grid=(N_PAD // BN,),
        in_specs=[
            pl.BlockSpec((NQ, BN, QC), lambda i: (0, i, 0)),
            pl.BlockSpec((1, DIM_H), lambda i: (0, 0)),
            pl.BlockSpec((DIM_H, DIM_OUT), lambda i: (0, 0)),
            pl.BlockSpec((1, DIM_OUT), lambda i: (0, 0)),
            pl.BlockSpec((NW, BN), lambda i: (0, i)),
        ],
        out_specs=pl.BlockSpec((BN, DIM_OUT), lambda i: (i, 0)),
        out_shape=jax.ShapeDtypeStruct((N_PAD, DIM_OUT), jnp.float32),
    )(agg, b2r, w3, b3r, deg_parts)


def kernel(x, edge_index, W1, b1, W2, b2, W3, b3):
    e = edge_index.shape[1]
    src = edge_index[0].astype(jnp.int32)
    dst = edge_index[1].astype(jnp.int32)

    eh = e // NW
    deg_parts = _make_hist(eh)(dst.reshape(NW, eh))

    em = -(-e // (NS * ESUP)) * ESUP
    pad = NS * em - e
    fill = jnp.full((pad,), N_NODES, jnp.int32)
    srcp = jnp.concatenate([src, fill]).reshape(NS, em // ESUP, KSUP, CB)
    dstp = jnp.concatenate([dst, fill]).reshape(NS, em // ESUP, KSUP, CB)
    mp = _make_mp(em // ESUP)

    xp = jnp.pad(x, ((0, N_PAD - N_NODES), (0, 0)))
    b1r = b1.reshape(1, DIM_H)
    b2r = b2.reshape(1, DIM_H)
    b3r = b3.reshape(1, DIM_OUT)

    hs1 = _lin1_call(xp, W1, deg_parts)
    agg1 = mp(hs1, srcp, dstp)
    hs2 = _mid_call(agg1, W2, b1r, deg_parts)
    agg2 = mp(hs2, srcp, dstp)
    return _out_call(agg2, b2r, W3, b3r, deg_parts)[:N_NODES]


# speedup vs baseline: 24.4496x; 1.2701x over previous
import functools

import jax
import jax.numpy as jnp
from jax import lax
from jax.experimental import pallas as pl
from jax.experimental.pallas import tpu as pltpu
from jax.experimental.pallas import tpu_sc as plsc

N_NODES = 10000
DIM_IN = 128
DIM_H = 256
DIM_OUT = 64

NC = 2
NS = 16
NW = NC * NS
L = 16

N_PAD = 10240
ROWS_PER_TILE = N_PAD // NS
NQ = 4
QC = DIM_H // NQ
CB = 128


def _hist_body(eh, dst_hbm, out_hbm, dst_v, hist_v):
    c = lax.axis_index("c")
    s = lax.axis_index("s")
    wid = s * NC + c
    pltpu.sync_copy(dst_hbm.at[wid], dst_v)
    zeros16 = jnp.zeros((L,), jnp.float32)

    def zbody(g, carry):
        hist_v[pl.ds(g * L, L)] = zeros16
        return carry

    lax.fori_loop(0, N_PAD // L, zbody, 0)
    ones16 = jnp.ones((L,), jnp.float32)

    def body(g, carry):
        idx = dst_v[pl.ds(g * L, L)]
        plsc.addupdate_scatter(hist_v, [idx], ones16)
        return carry

    lax.fori_loop(0, eh // L, body, 0)
    pltpu.sync_copy(hist_v, out_hbm.at[wid])


def _make_hist(eh):
    return pl.kernel(
        functools.partial(_hist_body, eh),
        out_type=jax.ShapeDtypeStruct((NW, N_PAD), jnp.float32),
        mesh=plsc.VectorSubcoreMesh(core_axis_name="c", subcore_axis_name="s"),
        compiler_params=pltpu.CompilerParams(needs_layout_passes=False),
        scratch_types=[
            pltpu.VMEM((eh,), jnp.int32),
            pltpu.VMEM((N_PAD,), jnp.float32),
        ],
    )


KSUP = 16
ESUP = KSUP * CB
NBUF = 4
GAHEAD = 3


def _mp_body(nsup, nqpc, hs_hbm, src_hbm, dst_hbm, out_hbm,
             src_buf, dst_buf, rows_v, hs_sp, agg_sp, *sems):
    gsem = sems[:NBUF]
    ssem = sems[NBUF:]
    c = lax.axis_index("c")
    s = lax.axis_index("s")
    r0 = s * ROWS_PER_TILE
    nblk = nsup * KSUP

    def idx_ref(buf, k):
        return buf.at[(k // KSUP) % 2, k % KSUP]

    def issue_gather(kg, bg):
        pltpu.async_copy(hs_sp.at[idx_ref(src_buf, kg)], rows_v.at[bg],
                         gsem[bg])

    for p in range(nqpc):
        q = c * nqpc + p
        seeds = [
            (hs_hbm.at[q, pl.ds(r0, ROWS_PER_TILE)],
             hs_sp.at[pl.ds(r0, ROWS_PER_TILE)], gsem[0]),
            (hs_hbm.at[q, pl.ds(r0, ROWS_PER_TILE)],
             agg_sp.at[pl.ds(r0, ROWS_PER_TILE)], gsem[1]),
            (src_hbm.at[s, 0], src_buf.at[0], ssem[0]),
            (dst_hbm.at[s, 0], dst_buf.at[0], ssem[1]),
        ]
        for sref, dref, sem in seeds:
            pltpu.async_copy(sref, dref, sem)
        for sref, dref, sem in seeds:
            pltpu.make_async_copy(sref, dref, sem).wait()
        plsc.subcore_barrier()
        for k0 in range(GAHEAD):
            issue_gather(k0, k0)

        def chunk(k, b):
            pltpu.make_async_copy(hs_sp.at[idx_ref(src_buf, k)],
                                  rows_v.at[b], gsem[b]).wait()
            pltpu.async_copy(rows_v.at[b], agg_sp.at[idx_ref(dst_buf, k)],
                             ssem[b], add=True)
            kg = k + GAHEAD
            bg = (b + GAHEAD) % NBUF

            @pl.when(kg < nblk)
            def _():
                @pl.when((kg % KSUP == 0) & (kg // KSUP > 0))
                def _():
                    pltpu.sync_copy(src_hbm.at[s, kg // KSUP],
                                    src_buf.at[(kg // KSUP) % 2])
                    pltpu.sync_copy(dst_hbm.at[s, kg // KSUP],
                                    dst_buf.at[(kg // KSUP) % 2])

                @pl.when(kg >= NBUF)
                def _():
                    pltpu.make_async_copy(
                        rows_v.at[bg],
                        agg_sp.at[idx_ref(dst_buf, kg - NBUF)],
                        ssem[bg]).wait()

                issue_gather(kg, bg)

        def group(g, carry):
            for b in range(NBUF):
                chunk(NBUF * g + b, b)
            return carry

        lax.fori_loop(0, nblk // NBUF, group, 0)
        for d in range(NBUF):
            k = nblk - NBUF + d
            pltpu.make_async_copy(rows_v.at[k % NBUF],
                                  agg_sp.at[idx_ref(dst_buf, k)],
                                  ssem[k % NBUF]).wait()
        plsc.subcore_barrier()
        pltpu.sync_copy(agg_sp.at[pl.ds(r0, ROWS_PER_TILE)],
                        out_hbm.at[q, pl.ds(r0, ROWS_PER_TILE)])


def _make_mp(nsup, nqpc):
    return pl.kernel(
        functools.partial(_mp_body, nsup, nqpc),
        out_type=jax.ShapeDtypeStruct((NC * nqpc, N_PAD, QC), jnp.float32),
        mesh=plsc.VectorSubcoreMesh(core_axis_name="c", subcore_axis_name="s"),
        compiler_params=pltpu.CompilerParams(needs_layout_passes=False,
                                             use_tc_tiling_on_sc=False),
        scratch_types=[
            pltpu.VMEM((2, KSUP, CB), jnp.int32),
            pltpu.VMEM((2, KSUP, CB), jnp.int32),
            pltpu.VMEM((NBUF, CB, QC), jnp.float32),
            pltpu.VMEM_SHARED((N_PAD, QC), jnp.float32),
            pltpu.VMEM_SHARED((N_PAD, QC), jnp.float32),
        ] + [pltpu.SemaphoreType.DMA] * (2 * NBUF),
    )


def _dinv_from(deg_ref):
    dsum = jnp.sum(deg_ref[...], axis=0) + 1.0
    return lax.rsqrt(dsum)[:, None]


def _store_quarters(out_ref, hs):
    for i in range(NQ):
        out_ref[i] = hs[:, i * QC:(i + 1) * QC]


def _scale_tc(x_ref, deg_ref, out_ref):
    dinv = _dinv_from(deg_ref)
    xs = x_ref[...] * dinv
    out_ref[0] = xs[:, :QC]
    out_ref[1] = xs[:, QC:]


def _relu_quarters(agg_ref, b_ref, dinv):
    return [jnp.maximum(agg_ref[i] * dinv + b_ref[:, i * QC:(i + 1) * QC], 0.0)
            for i in range(NQ)]


def _mid_tc(aggx_ref, w1_ref, b1_ref, w2_ref, deg_ref, out_ref):
    dinv = _dinv_from(deg_ref)
    al = aggx_ref[0] * dinv
    ar = aggx_ref[1] * dinv
    h1 = (jnp.dot(al, w1_ref[:QC, :], preferred_element_type=jnp.float32)
          + jnp.dot(ar, w1_ref[QC:, :], preferred_element_type=jnp.float32))
    o1 = jnp.maximum(h1 + b1_ref[...], 0.0)
    h2 = jnp.dot(o1, w2_ref[...], preferred_element_type=jnp.float32)
    _store_quarters(out_ref, h2 * dinv)


def _out_tc(agg_ref, b2_ref, w3_ref, b3_ref, deg_ref, out_ref):
    dinv = _dinv_from(deg_ref)
    hq = _relu_quarters(agg_ref, b2_ref, dinv)
    logits = sum(jnp.dot(hq[i], w3_ref[i * QC:(i + 1) * QC, :],
                         preferred_element_type=jnp.float32)
                 for i in range(NQ)) + b3_ref[...]
    m = jnp.max(logits, axis=1, keepdims=True)
    sh = logits - m
    lse = jnp.log(jnp.sum(jnp.exp(sh), axis=1, keepdims=True))
    out_ref[...] = sh - lse


BN = 1024


def _scale_call(xp, deg_parts):
    return pl.pallas_call(
        _scale_tc,
        grid=(N_PAD // BN,),
        in_specs=[
            pl.BlockSpec((BN, DIM_IN), lambda i: (i, 0)),
            pl.BlockSpec((NW, BN), lambda i: (0, i)),
        ],
        out_specs=pl.BlockSpec((NC, BN, QC), lambda i: (0, i, 0)),
        out_shape=jax.ShapeDtypeStruct((NC, N_PAD, QC), jnp.float32),
    )(xp, deg_parts)


def _mid_call(aggx, w1, b1r, w2, deg_parts):
    return pl.pallas_call(
        _mid_tc,
        grid=(N_PAD // BN,),
        in_specs=[
            pl.BlockSpec((NC, BN, QC), lambda i: (0, i, 0)),
            pl.BlockSpec((DIM_IN, DIM_H), lambda i: (0, 0)),
            pl.BlockSpec((1, DIM_H), lambda i: (0, 0)),
            pl.BlockSpec((DIM_H, DIM_H), lambda i: (0, 0)),
            pl.BlockSpec((NW, BN), lambda i: (0, i)),
        ],
        out_specs=pl.BlockSpec((NQ, BN, QC), lambda i: (0, i, 0)),
        out_shape=jax.ShapeDtypeStruct((NQ, N_PAD, QC), jnp.float32),
    )(aggx, w1, b1r, w2, deg_parts)


def _out_call(agg, b2r, w3, b3r, deg_parts):
    return pl.pallas_call(
        _out_tc,
        grid=(N_PAD // BN,),
        in_specs=[
            pl.BlockSpec((NQ, BN, QC), lambda i: (0, i, 0)),
            pl.BlockSpec((1, DIM_H), lambda i: (0, 0)),
            pl.BlockSpec((DIM_H, DIM_OUT), lambda i: (0, 0)),
            pl.BlockSpec((1, DIM_OUT), lambda i: (0, 0)),
            pl.BlockSpec((NW, BN), lambda i: (0, i)),
        ],
        out_specs=pl.BlockSpec((BN, DIM_OUT), lambda i: (i, 0)),
        out_shape=jax.ShapeDtypeStruct((N_PAD, DIM_OUT), jnp.float32),
    )(agg, b2r, w3, b3r, deg_parts)


def kernel(x, edge_index, W1, b1, W2, b2, W3, b3):
    e = edge_index.shape[1]
    src = edge_index[0].astype(jnp.int32)
    dst = edge_index[1].astype(jnp.int32)

    eh = e // NW
    deg_parts = _make_hist(eh)(dst.reshape(NW, eh))

    em = -(-e // (NS * ESUP)) * ESUP
    pad = NS * em - e
    fill = jnp.full((pad,), N_NODES, jnp.int32)
    srcp = jnp.concatenate([src, fill]).reshape(NS, em // ESUP, KSUP, CB)
    dstp = jnp.concatenate([dst, fill]).reshape(NS, em // ESUP, KSUP, CB)
    mp1 = _make_mp(em // ESUP, 1)
    mp2 = _make_mp(em // ESUP, NQ // NC)

    xp = jnp.pad(x, ((0, N_PAD - N_NODES), (0, 0)))
    b1r = b1.reshape(1, DIM_H)
    b2r = b2.reshape(1, DIM_H)
    b3r = b3.reshape(1, DIM_OUT)

    xs = _scale_call(xp, deg_parts)
    aggx = mp1(xs, srcp, dstp)
    hs2 = _mid_call(aggx, W1, b1r, W2, deg_parts)
    agg2 = mp2(hs2, srcp, dstp)
    return _out_call(agg2, b2r, W3, b3r, deg_parts)[:N_NODES]
